# Initial kernel scaffold; baseline (speedup 1.0000x reference)
import jax
import jax.numpy as jnp
from jax.experimental import pallas as pl


def kernel(embeddings, labels):
    raise NotImplementedError("write your pallas kernel here")


# speedup vs baseline: 1.3297x; 1.3297x over previous
import functools

import numpy as np

import jax
import jax.numpy as jnp
from jax.experimental import pallas as pl
from jax.experimental.pallas import tpu as pltpu


def _l2_normalize_kernel(x_ref, o_ref):
    x = x_ref[...].astype(jnp.float32)
    ss = jnp.sum(x * x, axis=-1, keepdims=True)
    o_ref[...] = (x * jax.lax.rsqrt(jnp.maximum(ss, 1e-24))).astype(o_ref.dtype)


def _pair_tile_kernel(ii_ref, jj_ref, ei_ref, ej_ref, li_ref, lj_ref,
                      ps_ref, pc_ref, ns_ref, nc_ref,
                      *, tile, chunk, n_valid, padded, n_blocks):
    tidx = pl.program_id(0)
    ib = ii_ref[tidx]
    jb = jj_ref[tidx]

    is_diag = ib == jb
    if padded:
        last = n_blocks - 1
        is_special = jnp.logical_or(
            is_diag, jnp.logical_or(ib == last, jb == last))
    else:
        is_special = is_diag

    ei = ei_ref[...]
    nck = tile // chunk

    def _gram_chunk(k):
        ejk = ej_ref[pl.ds(k * chunk, chunk), :]
        return jax.lax.dot_general(
            ei, ejk, dimension_numbers=(((1,), (1,)), ((), ())),
            preferred_element_type=jnp.float32)

    def _epilogue(k, masked):
        g = _gram_chunk(k)
        d = jnp.sqrt(jnp.maximum(2.0 - 2.0 * g, 0.0))
        nv = jnp.maximum(1.0 - d, 0.0)
        same = li_ref[...] == lj_ref[0:1, pl.ds(k * chunk, chunk)]
        if masked:
            r = ib * tile + jax.lax.broadcasted_iota(
                jnp.int32, (tile, chunk), 0)
            c = jb * tile + k * chunk + jax.lax.broadcasted_iota(
                jnp.int32, (tile, chunk), 1)
            pos_mask = jnp.logical_and(same, r != c)
            neg_mask = jnp.logical_not(same)
            if padded:
                valid = jnp.logical_and(r < n_valid, c < n_valid)
                pos_mask = jnp.logical_and(pos_mask, valid)
                neg_mask = jnp.logical_and(neg_mask, valid)
        else:
            pos_mask = same
            neg_mask = jnp.logical_not(same)
        pv = jnp.where(pos_mask, d, 0.0)
        nvm = jnp.where(neg_mask, nv, 0.0)
        sl = pl.ds(k * chunk, chunk)
        ps_ref[0, 0:1, sl] = jnp.sum(pv, axis=0, keepdims=True)
        pc_ref[0, 0:1, sl] = jnp.sum(
            (pv > 0.0).astype(jnp.float32), axis=0, keepdims=True)
        ns_ref[0, 0:1, sl] = jnp.sum(nvm, axis=0, keepdims=True)
        nc_ref[0, 0:1, sl] = jnp.sum(
            (nvm > 0.0).astype(jnp.float32), axis=0, keepdims=True)

    @pl.when(jnp.logical_not(is_special))
    def _fast():
        for k in range(nck):
            _epilogue(k, masked=False)

    @pl.when(is_special)
    def _slow():
        for k in range(nck):
            _epilogue(k, masked=True)


def kernel(embeddings, labels):
    n, dim = embeddings.shape
    if n % 1024 == 0:
        t = 1024
    elif n % 512 == 0:
        t = 512
    else:
        t = 512
    n_pad = -(-n // t) * t
    padded = n_pad != n

    x = embeddings.astype(jnp.float32)
    lab = labels.astype(jnp.int32)
    if padded:
        x = jnp.pad(x, ((0, n_pad - n), (0, 0)))
        lab = jnp.pad(lab, (0, n_pad - n), constant_values=-1)

    tn = 512
    e = pl.pallas_call(
        _l2_normalize_kernel,
        out_shape=jax.ShapeDtypeStruct((n_pad, dim), jnp.bfloat16),
        grid=(n_pad // tn,),
        in_specs=[pl.BlockSpec((tn, dim), lambda i: (i, 0))],
        out_specs=pl.BlockSpec((tn, dim), lambda i: (i, 0)),
        compiler_params=pltpu.CompilerParams(
            dimension_semantics=("parallel",)),
    )(x)

    lab_col = lab.reshape(n_pad, 1)
    lab_row = lab.reshape(1, n_pad)

    gi = n_pad // t
    tri = [(i, j) for i in range(gi) for j in range(i, gi)]
    ntiles = len(tri)
    ii = jnp.asarray(np.array([p[0] for p in tri], dtype=np.int32))
    jj = jnp.asarray(np.array([p[1] for p in tri], dtype=np.int32))

    chunk = 256 if t % 256 == 0 else t
    _pair_kernel = functools.partial(
        _pair_tile_kernel, tile=t, chunk=chunk, n_valid=n, padded=padded,
        n_blocks=gi)

    part_shape = jax.ShapeDtypeStruct((ntiles, 1, t), jnp.float32)
    part_spec = pl.BlockSpec((1, 1, t), lambda tt, ii_r, jj_r: (tt, 0, 0))

    ps, pc, ns, nc = pl.pallas_call(
        _pair_kernel,
        out_shape=(part_shape, part_shape, part_shape, part_shape),
        grid_spec=pltpu.PrefetchScalarGridSpec(
            num_scalar_prefetch=2,
            grid=(ntiles,),
            in_specs=[
                pl.BlockSpec((t, dim), lambda tt, ii_r, jj_r: (ii_r[tt], 0)),
                pl.BlockSpec((t, dim), lambda tt, ii_r, jj_r: (jj_r[tt], 0)),
                pl.BlockSpec((t, 1), lambda tt, ii_r, jj_r: (ii_r[tt], 0)),
                pl.BlockSpec((1, t), lambda tt, ii_r, jj_r: (0, jj_r[tt])),
            ],
            out_specs=(part_spec, part_spec, part_spec, part_spec),
        ),
        compiler_params=pltpu.CompilerParams(
            dimension_semantics=("parallel",),
            vmem_limit_bytes=56 * 1024 * 1024),
        cost_estimate=pl.CostEstimate(
            flops=2 * ntiles * t * t * dim,
            transcendentals=ntiles * t * t,
            bytes_accessed=(ntiles + gi) * t * dim * 2
                           + 4 * ntiles * t * 4 + 2 * n_pad * 4),
    )(ii, jj, e, e, lab_col, lab_row)

    w = jnp.where(ii == jj, 1.0, 2.0).astype(jnp.float32).reshape(ntiles, 1, 1)
    pos_sum = jnp.sum(ps * w)
    neg_sum = jnp.sum(ns * w)
    pos_cnt = jnp.sum((pc * w).astype(jnp.int32))
    neg_cnt = jnp.sum((nc * w).astype(jnp.int32))

    pos_term = jnp.where(
        pos_cnt > 0,
        pos_sum / jnp.maximum(pos_cnt.astype(jnp.float32), 1.0), 0.0)
    neg_term = jnp.where(
        neg_cnt > 0,
        neg_sum / jnp.maximum(neg_cnt.astype(jnp.float32), 1.0), 0.0)
    return pos_term + neg_term


# speedup vs baseline: 1.5448x; 1.1617x over previous
import functools

import numpy as np

import jax
import jax.numpy as jnp
from jax.experimental import pallas as pl
from jax.experimental.pallas import tpu as pltpu


def _l2_normalize_kernel(x_ref, o_ref):
    x = x_ref[...].astype(jnp.float32)
    ss = jnp.sum(x * x, axis=-1, keepdims=True)
    o_ref[...] = (x * jax.lax.rsqrt(
        jnp.maximum(ss, 1e-24) * 0.5)).astype(o_ref.dtype)


def _pair_tile_kernel(ii_ref, jj_ref, ei_ref, ej_ref, li_ref, lj_ref,
                      ps_ref, pc_ref, ns_ref, nc_ref,
                      *, tile, chunk, n_valid, padded, n_blocks):
    tidx = pl.program_id(0)
    ib = ii_ref[tidx]
    jb = jj_ref[tidx]

    is_diag = ib == jb
    if padded:
        last = n_blocks - 1
        is_special = jnp.logical_or(
            is_diag, jnp.logical_or(ib == last, jb == last))
    else:
        is_special = is_diag

    ei = ei_ref[...]
    nck = tile // chunk

    def _gram_chunk(k):
        ejk = ej_ref[pl.ds(k * chunk, chunk), :]
        return jax.lax.dot_general(
            ei, ejk, dimension_numbers=(((1,), (1,)), ((), ())),
            preferred_element_type=jnp.float32)

    def _epilogue(k, masked):
        x = 2.0 - _gram_chunk(k)
        xm = jnp.maximum(x, 0.0)
        d = xm * jax.lax.rsqrt(jnp.maximum(x, 1e-30))
        same = li_ref[...] == lj_ref[0:1, pl.ds(k * chunk, chunk)]
        if masked:
            r = ib * tile + jax.lax.broadcasted_iota(
                jnp.int32, (tile, chunk), 0)
            c = jb * tile + k * chunk + jax.lax.broadcasted_iota(
                jnp.int32, (tile, chunk), 1)
            pos_mask = jnp.logical_and(same, r != c)
            neg_mask = jnp.logical_not(same)
            if padded:
                valid = jnp.logical_and(r < n_valid, c < n_valid)
                pos_mask = jnp.logical_and(pos_mask, valid)
                neg_mask = jnp.logical_and(neg_mask, valid)
        else:
            pos_mask = same
            neg_mask = jnp.logical_not(same)
        pv = jnp.where(pos_mask, d, 0.0)
        nvm = jnp.where(neg_mask, jnp.maximum(1.0 - d, 0.0), 0.0)
        pi = jnp.minimum(pv * 1e38, 1.0)
        ni = jnp.minimum(nvm * 1e38, 1.0)
        sl = pl.ds(k * chunk, chunk)
        ps_ref[0, 0:1, sl] = jnp.sum(pv, axis=0, keepdims=True)
        pc_ref[0, 0:1, sl] = jnp.sum(pi, axis=0, keepdims=True)
        ns_ref[0, 0:1, sl] = jnp.sum(nvm, axis=0, keepdims=True)
        nc_ref[0, 0:1, sl] = jnp.sum(ni, axis=0, keepdims=True)

    @pl.when(jnp.logical_not(is_special))
    def _fast():
        for k in range(nck):
            _epilogue(k, masked=False)

    @pl.when(is_special)
    def _slow():
        for k in range(nck):
            _epilogue(k, masked=True)


def kernel(embeddings, labels):
    n, dim = embeddings.shape
    if n % 1024 == 0:
        t = 1024
    elif n % 512 == 0:
        t = 512
    else:
        t = 512
    n_pad = -(-n // t) * t
    padded = n_pad != n

    x = embeddings.astype(jnp.float32)
    lab = labels.astype(jnp.int32)
    if padded:
        x = jnp.pad(x, ((0, n_pad - n), (0, 0)))
        lab = jnp.pad(lab, (0, n_pad - n), constant_values=-1)

    tn = 512
    e = pl.pallas_call(
        _l2_normalize_kernel,
        out_shape=jax.ShapeDtypeStruct((n_pad, dim), jnp.bfloat16),
        grid=(n_pad // tn,),
        in_specs=[pl.BlockSpec((tn, dim), lambda i: (i, 0))],
        out_specs=pl.BlockSpec((tn, dim), lambda i: (i, 0)),
        compiler_params=pltpu.CompilerParams(
            dimension_semantics=("parallel",)),
    )(x)

    lab_col = lab.reshape(n_pad, 1)
    lab_row = lab.reshape(1, n_pad)

    gi = n_pad // t
    tri = [(i, j) for i in range(gi) for j in range(i, gi)]
    ntiles = len(tri)
    ii = jnp.asarray(np.array([p[0] for p in tri], dtype=np.int32))
    jj = jnp.asarray(np.array([p[1] for p in tri], dtype=np.int32))

    chunk = 256 if t % 256 == 0 else t
    _pair_kernel = functools.partial(
        _pair_tile_kernel, tile=t, chunk=chunk, n_valid=n, padded=padded,
        n_blocks=gi)

    part_shape = jax.ShapeDtypeStruct((ntiles, 1, t), jnp.float32)
    part_spec = pl.BlockSpec((1, 1, t), lambda tt, ii_r, jj_r: (tt, 0, 0))

    ps, pc, ns, nc = pl.pallas_call(
        _pair_kernel,
        out_shape=(part_shape, part_shape, part_shape, part_shape),
        grid_spec=pltpu.PrefetchScalarGridSpec(
            num_scalar_prefetch=2,
            grid=(ntiles,),
            in_specs=[
                pl.BlockSpec((t, dim), lambda tt, ii_r, jj_r: (ii_r[tt], 0)),
                pl.BlockSpec((t, dim), lambda tt, ii_r, jj_r: (jj_r[tt], 0)),
                pl.BlockSpec((t, 1), lambda tt, ii_r, jj_r: (ii_r[tt], 0)),
                pl.BlockSpec((1, t), lambda tt, ii_r, jj_r: (0, jj_r[tt])),
            ],
            out_specs=(part_spec, part_spec, part_spec, part_spec),
        ),
        compiler_params=pltpu.CompilerParams(
            dimension_semantics=("parallel",),
            vmem_limit_bytes=56 * 1024 * 1024),
        cost_estimate=pl.CostEstimate(
            flops=2 * ntiles * t * t * dim,
            transcendentals=ntiles * t * t,
            bytes_accessed=(ntiles + gi) * t * dim * 2
                           + 4 * ntiles * t * 4 + 2 * n_pad * 4),
    )(ii, jj, e, e, lab_col, lab_row)

    w = jnp.where(ii == jj, 1.0, 2.0).astype(jnp.float32).reshape(ntiles, 1, 1)
    pos_sum = jnp.sum(ps * w)
    neg_sum = jnp.sum(ns * w)
    pos_cnt = jnp.sum((pc * w).astype(jnp.int32))
    neg_cnt = jnp.sum((nc * w).astype(jnp.int32))

    pos_term = jnp.where(
        pos_cnt > 0,
        pos_sum / jnp.maximum(pos_cnt.astype(jnp.float32), 1.0), 0.0)
    neg_term = jnp.where(
        neg_cnt > 0,
        neg_sum / jnp.maximum(neg_cnt.astype(jnp.float32), 1.0), 0.0)
    return pos_term + neg_term


# speedup vs baseline: 1.5686x; 1.0154x over previous
import functools

import numpy as np

import jax
import jax.numpy as jnp
from jax.experimental import pallas as pl
from jax.experimental.pallas import tpu as pltpu


def _l2_normalize_kernel(x_ref, o_ref):
    x = x_ref[...].astype(jnp.float32)
    ss = jnp.sum(x * x, axis=-1, keepdims=True)
    o_ref[...] = (x * jax.lax.rsqrt(
        jnp.maximum(ss, 1e-24) * 0.5)).astype(o_ref.dtype)


def _pair_tile_kernel(ii_ref, jj_ref, ei_ref, ej_ref, li_ref, lj_ref,
                      ps_ref, pc_ref, ns_ref, nc_ref,
                      *, tile, chunk, n_valid, padded, n_blocks):
    tidx = pl.program_id(0)
    ib = ii_ref[tidx]
    jb = jj_ref[tidx]

    is_diag = ib == jb
    if padded:
        last = n_blocks - 1
        is_special = jnp.logical_or(
            is_diag, jnp.logical_or(ib == last, jb == last))
    else:
        is_special = is_diag

    ei = ei_ref[...]
    nck = tile // chunk

    def _gram_chunk(k):
        ejk = ej_ref[pl.ds(k * chunk, chunk), :]
        return jax.lax.dot_general(
            ei, ejk, dimension_numbers=(((1,), (1,)), ((), ())),
            preferred_element_type=jnp.float32)

    def _epilogue(k, masked):
        x = 2.0 - _gram_chunk(k)
        xm = jnp.maximum(x, 0.0)
        d = xm * jax.lax.rsqrt(jnp.maximum(x, 1e-30))
        same = li_ref[...] == lj_ref[0:1, pl.ds(k * chunk, chunk)]
        if masked:
            r = ib * tile + jax.lax.broadcasted_iota(
                jnp.int32, (tile, chunk), 0)
            c = jb * tile + k * chunk + jax.lax.broadcasted_iota(
                jnp.int32, (tile, chunk), 1)
            pos_mask = jnp.logical_and(same, r != c)
            neg_mask = jnp.logical_not(same)
            if padded:
                valid = jnp.logical_and(r < n_valid, c < n_valid)
                pos_mask = jnp.logical_and(pos_mask, valid)
                neg_mask = jnp.logical_and(neg_mask, valid)
        else:
            pos_mask = same
            neg_mask = jnp.logical_not(same)
        pv = jnp.where(pos_mask, d, 0.0)
        nvm = jnp.where(neg_mask, jnp.maximum(1.0 - d, 0.0), 0.0)
        pi = jnp.minimum(pv * 1e38, 1.0)
        ni = jnp.minimum(nvm * 1e38, 1.0)
        sl = pl.ds(k * chunk, chunk)
        ps_ref[0, 0:1, sl] = jnp.sum(pv, axis=0, keepdims=True)
        pc_ref[0, 0:1, sl] = jnp.sum(pi, axis=0, keepdims=True)
        ns_ref[0, 0:1, sl] = jnp.sum(nvm, axis=0, keepdims=True)
        nc_ref[0, 0:1, sl] = jnp.sum(ni, axis=0, keepdims=True)

    @pl.when(jnp.logical_not(is_special))
    def _fast():
        for k in range(nck):
            _epilogue(k, masked=False)

    @pl.when(is_special)
    def _slow():
        for k in range(nck):
            _epilogue(k, masked=True)


def kernel(embeddings, labels):
    n, dim = embeddings.shape
    if n % 1024 == 0:
        t = 1024
    elif n % 512 == 0:
        t = 512
    else:
        t = 512
    n_pad = -(-n // t) * t
    padded = n_pad != n

    x = embeddings.astype(jnp.float32)
    lab = labels.astype(jnp.int32)
    if padded:
        x = jnp.pad(x, ((0, n_pad - n), (0, 0)))
        lab = jnp.pad(lab, (0, n_pad - n), constant_values=-1)

    tn = 512
    e = pl.pallas_call(
        _l2_normalize_kernel,
        out_shape=jax.ShapeDtypeStruct((n_pad, dim), jnp.bfloat16),
        grid=(n_pad // tn,),
        in_specs=[pl.BlockSpec((tn, dim), lambda i: (i, 0))],
        out_specs=pl.BlockSpec((tn, dim), lambda i: (i, 0)),
        compiler_params=pltpu.CompilerParams(
            dimension_semantics=("parallel",)),
    )(x)

    lab_col = lab.reshape(n_pad, 1)
    lab_row = lab.reshape(1, n_pad)

    gi = n_pad // t
    tri = [(i, j) for i in range(gi) for j in range(i, gi)]
    ntiles = len(tri)
    ii = jnp.asarray(np.array([p[0] for p in tri], dtype=np.int32))
    jj = jnp.asarray(np.array([p[1] for p in tri], dtype=np.int32))

    chunk = 256 if t % 256 == 0 else t
    _pair_kernel = functools.partial(
        _pair_tile_kernel, tile=t, chunk=chunk, n_valid=n, padded=padded,
        n_blocks=gi)

    part_shape = jax.ShapeDtypeStruct((ntiles, 1, t), jnp.float32)
    part_spec = pl.BlockSpec((1, 1, t), lambda tt, ii_r, jj_r: (tt, 0, 0))

    ps, pc, ns, nc = pl.pallas_call(
        _pair_kernel,
        out_shape=(part_shape, part_shape, part_shape, part_shape),
        grid_spec=pltpu.PrefetchScalarGridSpec(
            num_scalar_prefetch=2,
            grid=(ntiles,),
            in_specs=[
                pl.BlockSpec((t, dim), lambda tt, ii_r, jj_r: (ii_r[tt], 0)),
                pl.BlockSpec((t, dim), lambda tt, ii_r, jj_r: (jj_r[tt], 0)),
                pl.BlockSpec((t, 1), lambda tt, ii_r, jj_r: (ii_r[tt], 0)),
                pl.BlockSpec((1, t), lambda tt, ii_r, jj_r: (0, jj_r[tt])),
            ],
            out_specs=(part_spec, part_spec, part_spec, part_spec),
        ),
        compiler_params=pltpu.CompilerParams(
            dimension_semantics=("parallel",),
            vmem_limit_bytes=28 * 1024 * 1024),
        cost_estimate=pl.CostEstimate(
            flops=2 * ntiles * t * t * dim,
            transcendentals=ntiles * t * t,
            bytes_accessed=(ntiles + gi) * t * dim * 2
                           + 4 * ntiles * t * 4 + 2 * n_pad * 4),
    )(ii, jj, e, e, lab_col, lab_row)

    w = jnp.where(ii == jj, 1.0, 2.0).astype(jnp.float32).reshape(ntiles, 1, 1)
    pos_sum = jnp.sum(ps * w)
    neg_sum = jnp.sum(ns * w)
    pos_cnt = jnp.sum((pc * w).astype(jnp.int32))
    neg_cnt = jnp.sum((nc * w).astype(jnp.int32))

    pos_term = jnp.where(
        pos_cnt > 0,
        pos_sum / jnp.maximum(pos_cnt.astype(jnp.float32), 1.0), 0.0)
    neg_term = jnp.where(
        neg_cnt > 0,
        neg_sum / jnp.maximum(neg_cnt.astype(jnp.float32), 1.0), 0.0)
    return pos_term + neg_term


# speedup vs baseline: 1.5686x; 1.0000x over previous
import functools

import jax
import jax.numpy as jnp
from jax.experimental import pallas as pl
from jax.experimental.pallas import tpu as pltpu


def _l2_normalize_kernel(x_ref, o_ref):
    x = x_ref[...].astype(jnp.float32)
    ss = jnp.sum(x * x, axis=-1, keepdims=True)
    o_ref[...] = (x * jax.lax.rsqrt(
        jnp.maximum(ss, 1e-24) * 0.5)).astype(o_ref.dtype)


def _trap_ij(r, k, gi):
    m = gi - 1 - r
    seg2 = k >= gi - r
    i = jnp.where(seg2, m, r)
    j = jnp.where(seg2, k - (gi - r) + m, r + k)
    return i, j


def _pair_tile_kernel(ei_ref, ej_ref, li_ref, lj_ref,
                      ps_ref, pc_ref, ns_ref, nc_ref,
                      *, tile, chunk, gi, n_valid, padded):
    r = pl.program_id(0)
    k = pl.program_id(1)
    ib, jb = _trap_ij(r, k, gi)

    is_diag = ib == jb
    if padded:
        last = gi - 1
        is_special = jnp.logical_or(
            is_diag, jnp.logical_or(ib == last, jb == last))
    else:
        is_special = is_diag

    ei = ei_ref[...]
    nck = tile // chunk

    def _gram_chunk(kk):
        ejk = ej_ref[pl.ds(kk * chunk, chunk), :]
        return jax.lax.dot_general(
            ei, ejk, dimension_numbers=(((1,), (1,)), ((), ())),
            preferred_element_type=jnp.float32)

    def _epilogue(kk, masked):
        x = 2.0 - _gram_chunk(kk)
        xm = jnp.maximum(x, 0.0)
        d = xm * jax.lax.rsqrt(jnp.maximum(x, 1e-30))
        same = li_ref[...] == lj_ref[0:1, pl.ds(kk * chunk, chunk)]
        if masked:
            rr = ib * tile + jax.lax.broadcasted_iota(
                jnp.int32, (tile, chunk), 0)
            cc = jb * tile + kk * chunk + jax.lax.broadcasted_iota(
                jnp.int32, (tile, chunk), 1)
            pos_mask = jnp.logical_and(same, rr != cc)
            neg_mask = jnp.logical_not(same)
            if padded:
                valid = jnp.logical_and(rr < n_valid, cc < n_valid)
                pos_mask = jnp.logical_and(pos_mask, valid)
                neg_mask = jnp.logical_and(neg_mask, valid)
        else:
            pos_mask = same
            neg_mask = jnp.logical_not(same)
        pv = jnp.where(pos_mask, d, 0.0)
        nvm = jnp.where(neg_mask, jnp.maximum(1.0 - d, 0.0), 0.0)
        pi = jnp.minimum(pv * 1e38, 1.0)
        ni = jnp.minimum(nvm * 1e38, 1.0)
        sl = pl.ds(kk * chunk, chunk)
        ps_ref[0, 0, 0:1, sl] = jnp.sum(pv, axis=0, keepdims=True)
        pc_ref[0, 0, 0:1, sl] = jnp.sum(pi, axis=0, keepdims=True)
        ns_ref[0, 0, 0:1, sl] = jnp.sum(nvm, axis=0, keepdims=True)
        nc_ref[0, 0, 0:1, sl] = jnp.sum(ni, axis=0, keepdims=True)

    @pl.when(jnp.logical_not(is_special))
    def _fast():
        for kk in range(nck):
            _epilogue(kk, masked=False)

    @pl.when(is_special)
    def _slow():
        for kk in range(nck):
            _epilogue(kk, masked=True)


def kernel(embeddings, labels):
    n, dim = embeddings.shape
    t = 1024 if n % 2048 == 0 else 512
    n_pad = -(-n // (2 * t)) * (2 * t)
    padded = n_pad != n

    x = embeddings.astype(jnp.float32)
    lab = labels.astype(jnp.int32)
    if padded:
        x = jnp.pad(x, ((0, n_pad - n), (0, 0)))
        lab = jnp.pad(lab, (0, n_pad - n), constant_values=-1)

    tn = 512
    e = pl.pallas_call(
        _l2_normalize_kernel,
        out_shape=jax.ShapeDtypeStruct((n_pad, dim), jnp.bfloat16),
        grid=(n_pad // tn,),
        in_specs=[pl.BlockSpec((tn, dim), lambda i: (i, 0))],
        out_specs=pl.BlockSpec((tn, dim), lambda i: (i, 0)),
        compiler_params=pltpu.CompilerParams(
            dimension_semantics=("parallel",)),
    )(x)

    lab_col = lab.reshape(n_pad, 1)
    lab_row = lab.reshape(1, n_pad)

    gi = n_pad // t
    gh = gi // 2
    gk = gi + 1

    chunk = 256 if t % 256 == 0 else t
    _pair_kernel = functools.partial(
        _pair_tile_kernel, tile=t, chunk=chunk, gi=gi, n_valid=n,
        padded=padded)

    def _ispec_i(r, k):
        i, _ = _trap_ij(r, k, gi)
        return (i, 0)

    def _ispec_j(r, k):
        _, j = _trap_ij(r, k, gi)
        return (j, 0)

    def _ispec_lj(r, k):
        _, j = _trap_ij(r, k, gi)
        return (0, j)

    part_shape = jax.ShapeDtypeStruct((gh, gk, 1, t), jnp.float32)
    part_spec = pl.BlockSpec((1, 1, 1, t), lambda r, k: (r, k, 0, 0))

    ps, pc, ns, nc = pl.pallas_call(
        _pair_kernel,
        out_shape=(part_shape, part_shape, part_shape, part_shape),
        grid=(gh, gk),
        in_specs=[
            pl.BlockSpec((t, dim), _ispec_i),
            pl.BlockSpec((t, dim), _ispec_j),
            pl.BlockSpec((t, 1), _ispec_i),
            pl.BlockSpec((1, t), _ispec_lj),
        ],
        out_specs=(part_spec, part_spec, part_spec, part_spec),
        compiler_params=pltpu.CompilerParams(
            dimension_semantics=("parallel", "arbitrary"),
            vmem_limit_bytes=28 * 1024 * 1024),
        cost_estimate=pl.CostEstimate(
            flops=2 * gh * gk * t * t * dim,
            transcendentals=gh * gk * t * t,
            bytes_accessed=(gh * gk + gi) * t * dim * 2
                           + 4 * gh * gk * t * 4 + 2 * n_pad * 4),
    )(e, e, lab_col, lab_row)

    rr = jnp.arange(gh, dtype=jnp.int32).reshape(gh, 1)
    kk = jnp.arange(gk, dtype=jnp.int32).reshape(1, gk)
    ii, jj = _trap_ij(rr, kk, gi)
    w = jnp.where(ii == jj, 1.0, 2.0).astype(jnp.float32).reshape(gh, gk, 1, 1)

    pos_sum = jnp.sum(ps * w)
    neg_sum = jnp.sum(ns * w)
    pos_cnt = jnp.sum((pc * w).astype(jnp.int32))
    neg_cnt = jnp.sum((nc * w).astype(jnp.int32))

    pos_term = jnp.where(
        pos_cnt > 0,
        pos_sum / jnp.maximum(pos_cnt.astype(jnp.float32), 1.0), 0.0)
    neg_term = jnp.where(
        neg_cnt > 0,
        neg_sum / jnp.maximum(neg_cnt.astype(jnp.float32), 1.0), 0.0)
    return pos_term + neg_term
